# Initial kernel scaffold; baseline (speedup 1.0000x reference)
import jax
import jax.numpy as jnp
from jax.experimental import pallas as pl


def kernel(feat, adj, W, a_l, b_l, a_r, b_r, bias):
    raise NotImplementedError("write your pallas kernel here")


# speedup vs baseline: 1.7602x; 1.7602x over previous
import functools

import jax
import jax.numpy as jnp
from jax.experimental import pallas as pl


def _proj_body(feat_ref, wt_ref, alt_ref, art_ref, bl_ref, br_ref,
               seq_ref, f1_ref, f2_ref):
    s = jnp.dot(feat_ref[...], wt_ref[...], preferred_element_type=jnp.float32)
    seq_ref[...] = s
    f1_ref[...] = jnp.dot(s, alt_ref[...], preferred_element_type=jnp.float32) + bl_ref[...]
    f2_ref[...] = jnp.dot(s, art_ref[...], preferred_element_type=jnp.float32) + br_ref[...]


def _attn_body(adj_ref, f1_ref, f2t_ref, seq_ref, bias_ref, out_ref):
    logits = f1_ref[...] + f2t_ref[...]
    lrelu = jnp.maximum(logits, 0.2 * logits)
    x = lrelu - 1e9 * (1.0 - adj_ref[...])
    m = jnp.max(x, axis=1, keepdims=True)
    e = jnp.exp(x - m)
    s = jnp.sum(e, axis=1, keepdims=True)
    v = jax.lax.dot_general(e, seq_ref[...], (((1,), (0,)), ((), ())),
                            preferred_element_type=jnp.float32)
    out = v / s + bias_ref[...]
    out_ref[...] = jnp.where(out > 0, out, jnp.exp(jnp.minimum(out, 0.0)) - 1.0)


@jax.jit
def kernel(feat, adj, W, a_l, b_l, a_r, b_r, bias):
    n, d_in = feat.shape
    d_out = W.shape[0]

    br1 = 2000
    seq, f1, f2 = pl.pallas_call(
        _proj_body,
        grid=(n // br1,),
        in_specs=[
            pl.BlockSpec((br1, d_in), lambda r: (r, 0)),
            pl.BlockSpec((d_in, d_out), lambda r: (0, 0)),
            pl.BlockSpec((d_out, 1), lambda r: (0, 0)),
            pl.BlockSpec((d_out, 1), lambda r: (0, 0)),
            pl.BlockSpec((1, 1), lambda r: (0, 0)),
            pl.BlockSpec((1, 1), lambda r: (0, 0)),
        ],
        out_specs=[
            pl.BlockSpec((br1, d_out), lambda r: (r, 0)),
            pl.BlockSpec((br1, 1), lambda r: (r, 0)),
            pl.BlockSpec((br1, 1), lambda r: (r, 0)),
        ],
        out_shape=[
            jax.ShapeDtypeStruct((n, d_out), jnp.float32),
            jax.ShapeDtypeStruct((n, 1), jnp.float32),
            jax.ShapeDtypeStruct((n, 1), jnp.float32),
        ],
    )(feat, W.T, a_l.T, a_r.T, b_l.reshape(1, 1), b_r.reshape(1, 1))

    f2t = f2.reshape(1, n)

    br = 200
    out = pl.pallas_call(
        _attn_body,
        grid=(n // br,),
        in_specs=[
            pl.BlockSpec((br, n), lambda r: (r, 0)),
            pl.BlockSpec((br, 1), lambda r: (r, 0)),
            pl.BlockSpec((1, n), lambda r: (0, 0)),
            pl.BlockSpec((n, d_out), lambda r: (0, 0)),
            pl.BlockSpec((1, d_out), lambda r: (0, 0)),
        ],
        out_specs=pl.BlockSpec((br, d_out), lambda r: (r, 0)),
        out_shape=jax.ShapeDtypeStruct((n, d_out), jnp.float32),
    )(adj, f1, f2t, seq, bias.reshape(1, d_out))

    return out


# speedup vs baseline: 1.8772x; 1.0665x over previous
import functools

import jax
import jax.numpy as jnp
from jax.experimental import pallas as pl


_LOG2E = 1.4426950408889634


def _proj_body(feat_ref, wt_ref, alt_ref, art_ref, bl_ref, br_ref,
               seq_ref, f1_ref, f2_ref):
    s = jnp.dot(feat_ref[...], wt_ref[...], preferred_element_type=jnp.float32)
    seq_ref[...] = s
    f1_ref[...] = (jnp.dot(s, alt_ref[...], preferred_element_type=jnp.float32)
                   + bl_ref[...]) * _LOG2E
    f2_ref[...] = (jnp.dot(s, art_ref[...], preferred_element_type=jnp.float32)
                   + br_ref[...]) * _LOG2E


def _attn_body(adj_ref, f1_ref, f2t_ref, seq_ref, bias_ref, out_ref):
    big = 1e9 * _LOG2E
    logits = f1_ref[...] + f2t_ref[...]
    lrelu = jnp.maximum(logits, 0.2 * logits)
    x = lrelu - big * (1.0 - adj_ref[...])
    m = jnp.max(x, axis=1, keepdims=True)
    e = jnp.exp2(x - m)
    s = jnp.sum(e, axis=1, keepdims=True)
    v = jax.lax.dot_general(e, seq_ref[...], (((1,), (0,)), ((), ())),
                            preferred_element_type=jnp.float32)
    out = v / s + bias_ref[...]
    out_ref[...] = jnp.where(out > 0, out, jnp.exp(jnp.minimum(out, 0.0)) - 1.0)


@jax.jit
def kernel(feat, adj, W, a_l, b_l, a_r, b_r, bias):
    n, d_in = feat.shape
    d_out = W.shape[0]

    br1 = 2000
    seq, f1, f2 = pl.pallas_call(
        _proj_body,
        grid=(n // br1,),
        in_specs=[
            pl.BlockSpec((br1, d_in), lambda r: (r, 0)),
            pl.BlockSpec((d_in, d_out), lambda r: (0, 0)),
            pl.BlockSpec((d_out, 1), lambda r: (0, 0)),
            pl.BlockSpec((d_out, 1), lambda r: (0, 0)),
            pl.BlockSpec((1, 1), lambda r: (0, 0)),
            pl.BlockSpec((1, 1), lambda r: (0, 0)),
        ],
        out_specs=[
            pl.BlockSpec((br1, d_out), lambda r: (r, 0)),
            pl.BlockSpec((br1, 1), lambda r: (r, 0)),
            pl.BlockSpec((br1, 1), lambda r: (r, 0)),
        ],
        out_shape=[
            jax.ShapeDtypeStruct((n, d_out), jnp.float32),
            jax.ShapeDtypeStruct((n, 1), jnp.float32),
            jax.ShapeDtypeStruct((n, 1), jnp.float32),
        ],
    )(feat, W.T, a_l.T, a_r.T, b_l.reshape(1, 1), b_r.reshape(1, 1))

    f2t = f2.reshape(1, n)

    br = 200
    out = pl.pallas_call(
        _attn_body,
        grid=(n // br,),
        in_specs=[
            pl.BlockSpec((br, n), lambda r: (r, 0)),
            pl.BlockSpec((br, 1), lambda r: (r, 0)),
            pl.BlockSpec((1, n), lambda r: (0, 0)),
            pl.BlockSpec((n, d_out), lambda r: (0, 0)),
            pl.BlockSpec((1, d_out), lambda r: (0, 0)),
        ],
        out_specs=pl.BlockSpec((br, d_out), lambda r: (r, 0)),
        out_shape=jax.ShapeDtypeStruct((n, d_out), jnp.float32),
    )(adj, f1, f2t, seq, bias.reshape(1, d_out))

    return out


# speedup vs baseline: 2.6132x; 1.3921x over previous
import functools

import jax
import jax.numpy as jnp
from jax import lax
from jax.experimental import pallas as pl

_LOG2E = 1.4426950408889634


def _proj_body(feat_ref, wt_ref, alt_ref, art_ref, bl_ref, br_ref,
               seqa_ref, f1_ref, f2_ref):
    s = jnp.dot(feat_ref[...], wt_ref[...], preferred_element_type=jnp.float32)
    br1, d = s.shape
    seqa_ref[:, :d] = s.astype(jnp.bfloat16)
    col = lax.broadcasted_iota(jnp.int32, (br1, d), 1)
    seqa_ref[:, d:] = jnp.where(col == 0, 1.0, 0.0).astype(jnp.bfloat16)
    f1_ref[...] = (jnp.dot(s, alt_ref[...], preferred_element_type=jnp.float32)
                   + bl_ref[...]) * _LOG2E
    f2_ref[...] = (jnp.dot(s, art_ref[...], preferred_element_type=jnp.float32)
                   + br_ref[...]) * _LOG2E


def _attn_body(adj_ref, f1_ref, f2t_ref, seqa_ref, bias_ref, out_ref):
    big = 1e9 * _LOG2E
    logits = f1_ref[...] + f2t_ref[...]
    lrelu = jnp.maximum(logits, 0.2 * logits)
    x = lrelu - big * (1.0 - adj_ref[...])
    m = jnp.max(x, axis=1, keepdims=True)
    e = jnp.exp2(x - m).astype(jnp.bfloat16)
    va = jax.lax.dot_general(e, seqa_ref[...], (((1,), (0,)), ((), ())),
                             preferred_element_type=jnp.float32)
    d = out_ref.shape[1]
    out = va[:, :d] / va[:, d:d + 1] + bias_ref[...]
    out_ref[...] = jnp.where(out > 0, out, jnp.exp(jnp.minimum(out, 0.0)) - 1.0)


@jax.jit
def kernel(feat, adj, W, a_l, b_l, a_r, b_r, bias):
    n, d_in = feat.shape
    d_out = W.shape[0]

    br1 = 2000
    seqa, f1, f2 = pl.pallas_call(
        _proj_body,
        grid=(n // br1,),
        in_specs=[
            pl.BlockSpec((br1, d_in), lambda r: (r, 0)),
            pl.BlockSpec((d_in, d_out), lambda r: (0, 0)),
            pl.BlockSpec((d_out, 1), lambda r: (0, 0)),
            pl.BlockSpec((d_out, 1), lambda r: (0, 0)),
            pl.BlockSpec((1, 1), lambda r: (0, 0)),
            pl.BlockSpec((1, 1), lambda r: (0, 0)),
        ],
        out_specs=[
            pl.BlockSpec((br1, 2 * d_out), lambda r: (r, 0)),
            pl.BlockSpec((br1, 1), lambda r: (r, 0)),
            pl.BlockSpec((br1, 1), lambda r: (r, 0)),
        ],
        out_shape=[
            jax.ShapeDtypeStruct((n, 2 * d_out), jnp.bfloat16),
            jax.ShapeDtypeStruct((n, 1), jnp.float32),
            jax.ShapeDtypeStruct((n, 1), jnp.float32),
        ],
    )(feat, W.T, a_l.T, a_r.T, b_l.reshape(1, 1), b_r.reshape(1, 1))

    f2t = f2.reshape(1, n)

    br = 200
    out = pl.pallas_call(
        _attn_body,
        grid=(n // br,),
        in_specs=[
            pl.BlockSpec((br, n), lambda r: (r, 0)),
            pl.BlockSpec((br, 1), lambda r: (r, 0)),
            pl.BlockSpec((1, n), lambda r: (0, 0)),
            pl.BlockSpec((n, 2 * d_out), lambda r: (0, 0)),
            pl.BlockSpec((1, d_out), lambda r: (0, 0)),
        ],
        out_specs=pl.BlockSpec((br, d_out), lambda r: (r, 0)),
        out_shape=jax.ShapeDtypeStruct((n, d_out), jnp.float32),
    )(adj, f1, f2t, seqa, bias.reshape(1, d_out))

    return out
